# Initial kernel scaffold; baseline (speedup 1.0000x reference)
import jax
import jax.numpy as jnp
from jax.experimental import pallas as pl


def kernel(x, edge_index, Wq, bq, Wk, bk, W1, W2, b2):
    raise NotImplementedError("write your pallas kernel here")


# speedup vs baseline: 11.9468x; 11.9468x over previous
import functools

import numpy as np
import jax
import jax.numpy as jnp
from jax import lax
from jax.experimental import pallas as pl
from jax.experimental.pallas import tpu as pltpu
from jax.experimental.pallas import tpu_sc as plsc


_LANES = 16
_CH = 128


def _proj_body(x_ref, wq_ref, bq_ref, wk_ref, bk_ref, q_ref, k_ref):
    xb = x_ref[...]
    q_ref[...] = jnp.tanh(
        jnp.dot(xb, wq_ref[...], preferred_element_type=jnp.float32) + bq_ref[...])
    k_ref[...] = (
        jnp.dot(xb, wk_ref[...], preferred_element_type=jnp.float32) + bk_ref[...])


def _combine_body(inv_sqrt_s, x_ref, q_ref, k_ref, a0_ref, a1_ref, d0_ref,
                  d1_ref, w1_ref, w2_ref, b2_ref, o_ref):
    xb = x_ref[...]
    ps = jnp.exp(jnp.sum(q_ref[...] * k_ref[...], axis=1, keepdims=True)
                 * inv_sqrt_s)
    den = d0_ref[...] + d1_ref[...] + ps
    att = (a0_ref[...] + a1_ref[...] + ps * xb) / den
    out = (jnp.dot(xb, w1_ref[...], preferred_element_type=jnp.float32)
           + jnp.dot(att, w2_ref[...], preferred_element_type=jnp.float32)
           + b2_ref[...])
    nrm = jnp.sqrt(jnp.sum(out * out, axis=1, keepdims=True))
    o_ref[...] = jnp.maximum(out / nrm, 0.0)


def _make_sc_kernel(N, D, S, E, Epad, inv_sqrt_s):
    NC, NS = 2, 16
    NW = NC * NS
    EPW = Epad // NW
    NCHUNK = EPW // _CH
    ROWS_PT = -(-N // (NS * 8)) * 8
    Npad = ROWS_PT * NS
    ZD = Npad // 8
    nz = D // _LANES

    def body(x_hbm, q_hbm, k_hbm, src_hbm, dst_hbm,
             att_out, den_out,
             src_v, dst_v, q_rows, k_rows, x_rows, w_v, zb_den,
             att_sh, den_sh):
        cid = lax.axis_index("c")
        sid = lax.axis_index("s")
        iota = lax.iota(jnp.int32, _LANES)
        zeros16 = jnp.zeros((_LANES,), jnp.float32)

        def _zero_row(i, c):
            for cc in range(nz):
                x_rows[i, pl.ds(cc * _LANES, _LANES)] = zeros16
            return c
        lax.fori_loop(0, _CH, _zero_row, 0)

        r0 = sid * ROWS_PT
        off = 0
        rem = ROWS_PT
        while rem > 0:
            n = min(rem, _CH)
            pltpu.sync_copy(x_rows.at[pl.ds(0, n)],
                            att_sh.at[pl.ds(r0 + off, n)])
            off += n
            rem -= n

        def _zero_den(i, c):
            zb_den[pl.ds(i * _LANES, _LANES)] = zeros16
            return c
        lax.fori_loop(0, ZD // _LANES, _zero_den, 0)

        @pl.when(sid == 0)
        def _():
            for o in range(0, Npad, ZD):
                pltpu.sync_copy(zb_den, den_sh.at[pl.ds(o, ZD)])

        plsc.subcore_barrier()

        wbase = (cid * NS + sid) * EPW

        def chunk(ch, c):
            base = wbase + ch * _CH
            pltpu.sync_copy(src_hbm.at[pl.ds(base, _CH)], src_v)
            pltpu.sync_copy(dst_hbm.at[pl.ds(base, _CH)], dst_v)
            pltpu.sync_copy(q_hbm.at[src_v], q_rows)
            pltpu.sync_copy(k_hbm.at[dst_v], k_rows)
            for g in range(_CH // _LANES):
                row_idx = g * _LANES + iota
                acc = jnp.zeros((_LANES,), jnp.float32)
                for j in range(S):
                    cj = jnp.full((_LANES,), j, jnp.int32)
                    qv = plsc.load_gather(q_rows, [row_idx, cj])
                    kv = plsc.load_gather(k_rows, [row_idx, cj])
                    acc = acc + qv * kv
                p = jnp.exp(acc * inv_sqrt_s)
                geid = base + row_idx
                p = jnp.where(geid < E, p, 0.0)
                w_v[pl.ds(g * _LANES, _LANES)] = p
            pltpu.sync_copy(w_v, den_sh.at[dst_v], add=True)
            pltpu.sync_copy(x_hbm.at[src_v], x_rows)

            def scale_grp(g, cc):
                wv = w_v[pl.ds(g * _LANES, _LANES)]
                for l in range(_LANES):
                    ws = wv[l]
                    r = g * _LANES + l
                    for cc2 in range(nz):
                        sl = pl.ds(cc2 * _LANES, _LANES)
                        x_rows[r, sl] = x_rows[r, sl] * ws
                return cc
            lax.fori_loop(0, _CH // _LANES, scale_grp, 0)
            pltpu.sync_copy(x_rows, att_sh.at[dst_v], add=True)
            return c

        lax.fori_loop(0, NCHUNK, chunk, 0)

        plsc.subcore_barrier()

        pltpu.sync_copy(att_sh.at[pl.ds(r0, ROWS_PT)],
                        att_out.at[cid, pl.ds(r0, ROWS_PT)])

        @pl.when(sid == 0)
        def _():
            pltpu.sync_copy(den_sh, den_out.at[cid])

    mesh = plsc.VectorSubcoreMesh(core_axis_name="c", subcore_axis_name="s")
    return pl.kernel(
        body,
        mesh=mesh,
        compiler_params=pltpu.CompilerParams(
            needs_layout_passes=False, use_tc_tiling_on_sc=False),
        out_type=[
            jax.ShapeDtypeStruct((NC, Npad, D), jnp.float32),
            jax.ShapeDtypeStruct((NC, Npad), jnp.float32),
        ],
        scratch_types=[
            pltpu.VMEM((_CH,), jnp.int32),
            pltpu.VMEM((_CH,), jnp.int32),
            pltpu.VMEM((_CH, _LANES), jnp.float32),
            pltpu.VMEM((_CH, _LANES), jnp.float32),
            pltpu.VMEM((_CH, D), jnp.float32),
            pltpu.VMEM((_CH,), jnp.float32),
            pltpu.VMEM((ZD,), jnp.float32),
            pltpu.VMEM_SHARED((Npad, D), jnp.float32),
            pltpu.VMEM_SHARED((Npad,), jnp.float32),
        ],
    )


def kernel(x, edge_index, Wq, bq, Wk, bk, W1, W2, b2):
    N, D = x.shape
    S = Wq.shape[0]
    E = edge_index.shape[1]
    inv_sqrt_s = np.float32(1.0 / np.sqrt(S))

    WqT = jnp.zeros((D, _LANES), jnp.float32).at[:, :S].set(Wq.T)
    WkT = jnp.zeros((D, _LANES), jnp.float32).at[:, :S].set(Wk.T)
    bq_p = jnp.zeros((1, _LANES), jnp.float32).at[0, :S].set(bq)
    bk_p = jnp.zeros((1, _LANES), jnp.float32).at[0, :S].set(bk)

    align = 32 * _CH
    Epad = ((E + align - 1) // align) * align
    pad = Epad - E
    src_p = jnp.concatenate([edge_index[0], jnp.zeros((pad,), edge_index.dtype)])
    dst_p = jnp.concatenate([edge_index[1], jnp.zeros((pad,), edge_index.dtype)])

    BA = 400
    grid_a = N // BA
    q_pad, k_pad = pl.pallas_call(
        _proj_body,
        grid=(grid_a,),
        in_specs=[
            pl.BlockSpec((BA, D), lambda i: (i, 0)),
            pl.BlockSpec((D, _LANES), lambda i: (0, 0)),
            pl.BlockSpec((1, _LANES), lambda i: (0, 0)),
            pl.BlockSpec((D, _LANES), lambda i: (0, 0)),
            pl.BlockSpec((1, _LANES), lambda i: (0, 0)),
        ],
        out_specs=[
            pl.BlockSpec((BA, _LANES), lambda i: (i, 0)),
            pl.BlockSpec((BA, _LANES), lambda i: (i, 0)),
        ],
        out_shape=[
            jax.ShapeDtypeStruct((N, _LANES), jnp.float32),
            jax.ShapeDtypeStruct((N, _LANES), jnp.float32),
        ],
    )(x, WqT, bq_p, WkT, bk_p)

    sc_fn = _make_sc_kernel(N, D, S, E, Epad, inv_sqrt_s)
    att_part, den_part = sc_fn(x, q_pad, k_pad, src_p, dst_p)

    a0, a1 = att_part[0, :N], att_part[1, :N]
    d0 = den_part[0, :N].reshape(N, 1)
    d1 = den_part[1, :N].reshape(N, 1)

    out = pl.pallas_call(
        functools.partial(_combine_body, inv_sqrt_s),
        grid=(grid_a,),
        in_specs=[
            pl.BlockSpec((BA, D), lambda i: (i, 0)),
            pl.BlockSpec((BA, _LANES), lambda i: (i, 0)),
            pl.BlockSpec((BA, _LANES), lambda i: (i, 0)),
            pl.BlockSpec((BA, D), lambda i: (i, 0)),
            pl.BlockSpec((BA, D), lambda i: (i, 0)),
            pl.BlockSpec((BA, 1), lambda i: (i, 0)),
            pl.BlockSpec((BA, 1), lambda i: (i, 0)),
            pl.BlockSpec((D, D), lambda i: (0, 0)),
            pl.BlockSpec((D, D), lambda i: (0, 0)),
            pl.BlockSpec((1, D), lambda i: (0, 0)),
        ],
        out_specs=pl.BlockSpec((BA, D), lambda i: (i, 0)),
        out_shape=jax.ShapeDtypeStruct((N, D), jnp.float32),
    )(x, q_pad, k_pad, a0, a1, d0, d1, W1.T, W2.T, b2.reshape(1, D))
    return out


# speedup vs baseline: 15.1593x; 1.2689x over previous
import functools

import numpy as np
import jax
import jax.numpy as jnp
from jax import lax
from jax.experimental import pallas as pl
from jax.experimental.pallas import tpu as pltpu
from jax.experimental.pallas import tpu_sc as plsc


_LANES = 16
_CH = 128


def _proj_body(x_ref, wq_ref, bq_ref, wk_ref, bk_ref, q_ref, k_ref):
    xb = x_ref[...]
    q_ref[...] = jnp.tanh(
        jnp.dot(xb, wq_ref[...], preferred_element_type=jnp.float32) + bq_ref[...])
    k_ref[...] = (
        jnp.dot(xb, wk_ref[...], preferred_element_type=jnp.float32) + bk_ref[...])


def _combine_body(inv_sqrt_s, x_ref, q_ref, k_ref, a0_ref, a1_ref, d0_ref,
                  d1_ref, w1_ref, w2_ref, b2_ref, o_ref):
    xb = x_ref[...]
    ps = jnp.exp(jnp.sum(q_ref[...] * k_ref[...], axis=1, keepdims=True)
                 * inv_sqrt_s)
    den = d0_ref[...] + d1_ref[...] + ps
    att = (a0_ref[...] + a1_ref[...] + ps * xb) / den
    out = (jnp.dot(xb, w1_ref[...], preferred_element_type=jnp.float32)
           + jnp.dot(att, w2_ref[...], preferred_element_type=jnp.float32)
           + b2_ref[...])
    nrm = jnp.sqrt(jnp.sum(out * out, axis=1, keepdims=True))
    o_ref[...] = jnp.maximum(out / nrm, 0.0)


def _make_sc_kernel(N, D, S, E, Epad, inv_sqrt_s):
    NC, NS = 2, 16
    NW = NC * NS
    EPW = Epad // NW
    NCHUNK = EPW // _CH
    ROWS_PT = -(-N // (NS * 8)) * 8
    Npad = ROWS_PT * NS
    ZD = Npad // 8
    nz = D // _LANES

    def body(x_hbm, q_hbm, k_hbm, sd_hbm,
             att_out, den_out,
             sd0, sd1, q0, q1, k0, k1, x0, x1, w_v, zb_den,
             att_sh, den_sh, sem0, sem1):
        cid = lax.axis_index("c")
        sid = lax.axis_index("s")
        iota = lax.iota(jnp.int32, _LANES)
        zeros16 = jnp.zeros((_LANES,), jnp.float32)

        def _zero_row(i, c):
            for cc in range(nz):
                x0[i, pl.ds(cc * _LANES, _LANES)] = zeros16
            return c
        lax.fori_loop(0, _CH, _zero_row, 0)

        r0 = sid * ROWS_PT
        off = 0
        rem = ROWS_PT
        while rem > 0:
            n = min(rem, _CH)
            pltpu.sync_copy(x0.at[pl.ds(0, n)],
                            att_sh.at[pl.ds(r0 + off, n)])
            off += n
            rem -= n

        def _zero_den(i, c):
            zb_den[pl.ds(i * _LANES, _LANES)] = zeros16
            return c
        lax.fori_loop(0, ZD // _LANES, _zero_den, 0)

        @pl.when(sid == 0)
        def _():
            for o in range(0, Npad, ZD):
                pltpu.sync_copy(zb_den, den_sh.at[pl.ds(o, ZD)])

        plsc.subcore_barrier()

        wid = cid * NS + sid
        cbase = wid * NCHUNK
        bufs = ((sd0, q0, k0, x0, sem0), (sd1, q1, k1, x1, sem1))

        def start_fetch(b, ch):
            sd_v, q_r, k_r, x_r, sem = bufs[b]
            pltpu.sync_copy(sd_hbm.at[cbase + ch], sd_v)
            pltpu.make_async_copy(q_hbm.at[sd_v.at[0]], q_r, sem).start()
            pltpu.make_async_copy(k_hbm.at[sd_v.at[1]], k_r, sem).start()
            pltpu.make_async_copy(x_hbm.at[sd_v.at[0]], x_r, sem).start()

        def process(b, ch):
            sd_v, q_r, k_r, x_r, sem = bufs[b]
            pltpu.make_async_copy(q_hbm.at[sd_v.at[0]], q_r, sem).wait()
            pltpu.make_async_copy(k_hbm.at[sd_v.at[1]], k_r, sem).wait()
            pltpu.make_async_copy(x_hbm.at[sd_v.at[0]], x_r, sem).wait()
            ebase = (cbase + ch) * _CH
            for g in range(_CH // _LANES):
                row_idx = g * _LANES + iota
                acc = jnp.zeros((_LANES,), jnp.float32)
                for j in range(S):
                    cj = jnp.full((_LANES,), j, jnp.int32)
                    qv = plsc.load_gather(q_r, [row_idx, cj])
                    kv = plsc.load_gather(k_r, [row_idx, cj])
                    acc = acc + qv * kv
                p = jnp.exp(acc * inv_sqrt_s)
                p = jnp.where(ebase + row_idx < E, p, 0.0)
                w_v[pl.ds(g * _LANES, _LANES)] = p
                for l in range(_LANES):
                    ws = p[l]
                    r = g * _LANES + l
                    for cc in range(nz):
                        sl = pl.ds(cc * _LANES, _LANES)
                        x_r[r, sl] = x_r[r, sl] * ws
            pltpu.sync_copy(w_v, den_sh.at[sd_v.at[1]], add=True)
            pltpu.sync_copy(x_r, att_sh.at[sd_v.at[1]], add=True)

        start_fetch(0, 0)
        start_fetch(1, 1)

        def ring(i, c):
            ch0 = i * 2
            process(0, ch0)

            @pl.when(ch0 + 2 < NCHUNK)
            def _():
                start_fetch(0, ch0 + 2)

            process(1, ch0 + 1)

            @pl.when(ch0 + 3 < NCHUNK)
            def _():
                start_fetch(1, ch0 + 3)
            return c

        lax.fori_loop(0, NCHUNK // 2, ring, 0)

        plsc.subcore_barrier()

        pltpu.sync_copy(att_sh.at[pl.ds(r0, ROWS_PT)],
                        att_out.at[cid, pl.ds(r0, ROWS_PT)])

        @pl.when(sid == 0)
        def _():
            pltpu.sync_copy(den_sh, den_out.at[cid])

    mesh = plsc.VectorSubcoreMesh(core_axis_name="c", subcore_axis_name="s")
    return pl.kernel(
        body,
        mesh=mesh,
        compiler_params=pltpu.CompilerParams(
            needs_layout_passes=False, use_tc_tiling_on_sc=False),
        out_type=[
            jax.ShapeDtypeStruct((NC, Npad, D), jnp.float32),
            jax.ShapeDtypeStruct((NC, Npad), jnp.float32),
        ],
        scratch_types=[
            pltpu.VMEM((2, _CH), jnp.int32),
            pltpu.VMEM((2, _CH), jnp.int32),
            pltpu.VMEM((_CH, _LANES), jnp.float32),
            pltpu.VMEM((_CH, _LANES), jnp.float32),
            pltpu.VMEM((_CH, _LANES), jnp.float32),
            pltpu.VMEM((_CH, _LANES), jnp.float32),
            pltpu.VMEM((_CH, D), jnp.float32),
            pltpu.VMEM((_CH, D), jnp.float32),
            pltpu.VMEM((_CH,), jnp.float32),
            pltpu.VMEM((ZD,), jnp.float32),
            pltpu.VMEM_SHARED((Npad, D), jnp.float32),
            pltpu.VMEM_SHARED((Npad,), jnp.float32),
            pltpu.SemaphoreType.DMA,
            pltpu.SemaphoreType.DMA,
        ],
    )


def kernel(x, edge_index, Wq, bq, Wk, bk, W1, W2, b2):
    N, D = x.shape
    S = Wq.shape[0]
    E = edge_index.shape[1]
    inv_sqrt_s = np.float32(1.0 / np.sqrt(S))

    WqT = jnp.zeros((D, _LANES), jnp.float32).at[:, :S].set(Wq.T)
    WkT = jnp.zeros((D, _LANES), jnp.float32).at[:, :S].set(Wk.T)
    bq_p = jnp.zeros((1, _LANES), jnp.float32).at[0, :S].set(bq)
    bk_p = jnp.zeros((1, _LANES), jnp.float32).at[0, :S].set(bk)

    align = 2 * 32 * _CH
    Epad = ((E + align - 1) // align) * align
    pad = Epad - E
    src_p = jnp.concatenate([edge_index[0], jnp.zeros((pad,), edge_index.dtype)])
    dst_p = jnp.concatenate([edge_index[1], jnp.zeros((pad,), edge_index.dtype)])
    sd = jnp.stack([src_p.reshape(Epad // _CH, _CH),
                    dst_p.reshape(Epad // _CH, _CH)], axis=1)

    BA = 400
    grid_a = N // BA
    q_pad, k_pad = pl.pallas_call(
        _proj_body,
        grid=(grid_a,),
        in_specs=[
            pl.BlockSpec((BA, D), lambda i: (i, 0)),
            pl.BlockSpec((D, _LANES), lambda i: (0, 0)),
            pl.BlockSpec((1, _LANES), lambda i: (0, 0)),
            pl.BlockSpec((D, _LANES), lambda i: (0, 0)),
            pl.BlockSpec((1, _LANES), lambda i: (0, 0)),
        ],
        out_specs=[
            pl.BlockSpec((BA, _LANES), lambda i: (i, 0)),
            pl.BlockSpec((BA, _LANES), lambda i: (i, 0)),
        ],
        out_shape=[
            jax.ShapeDtypeStruct((N, _LANES), jnp.float32),
            jax.ShapeDtypeStruct((N, _LANES), jnp.float32),
        ],
    )(x, WqT, bq_p, WkT, bk_p)

    sc_fn = _make_sc_kernel(N, D, S, E, Epad, inv_sqrt_s)
    att_part, den_part = sc_fn(x, q_pad, k_pad, sd)

    a0, a1 = att_part[0, :N], att_part[1, :N]
    d0 = den_part[0, :N].reshape(N, 1)
    d1 = den_part[1, :N].reshape(N, 1)

    out = pl.pallas_call(
        functools.partial(_combine_body, inv_sqrt_s),
        grid=(grid_a,),
        in_specs=[
            pl.BlockSpec((BA, D), lambda i: (i, 0)),
            pl.BlockSpec((BA, _LANES), lambda i: (i, 0)),
            pl.BlockSpec((BA, _LANES), lambda i: (i, 0)),
            pl.BlockSpec((BA, D), lambda i: (i, 0)),
            pl.BlockSpec((BA, D), lambda i: (i, 0)),
            pl.BlockSpec((BA, 1), lambda i: (i, 0)),
            pl.BlockSpec((BA, 1), lambda i: (i, 0)),
            pl.BlockSpec((D, D), lambda i: (0, 0)),
            pl.BlockSpec((D, D), lambda i: (0, 0)),
            pl.BlockSpec((1, D), lambda i: (0, 0)),
        ],
        out_specs=pl.BlockSpec((BA, D), lambda i: (i, 0)),
        out_shape=jax.ShapeDtypeStruct((N, D), jnp.float32),
    )(x, q_pad, k_pad, a0, a1, d0, d1, W1.T, W2.T, b2.reshape(1, D))
    return out


# speedup vs baseline: 21.8064x; 1.4385x over previous
import functools

import numpy as np
import jax
import jax.numpy as jnp
from jax import lax
from jax.experimental import pallas as pl
from jax.experimental.pallas import tpu as pltpu
from jax.experimental.pallas import tpu_sc as plsc


_LANES = 16
_CH = 128


def _proj_body(x_ref, wq_ref, bq_ref, wk_ref, bk_ref, q_ref, k_ref):
    xb = x_ref[...]
    q_ref[...] = jnp.tanh(
        jnp.dot(xb, wq_ref[...], preferred_element_type=jnp.float32) + bq_ref[...])
    k_ref[...] = (
        jnp.dot(xb, wk_ref[...], preferred_element_type=jnp.float32) + bk_ref[...])


def _combine_body(inv_sqrt_s, x_ref, q_ref, k_ref, a0_ref, a1_ref, d0_ref,
                  d1_ref, w1_ref, w2_ref, b2_ref, o_ref):
    xb = x_ref[...]
    ps = jnp.exp(jnp.sum(q_ref[...] * k_ref[...], axis=1, keepdims=True)
                 * inv_sqrt_s)
    den = d0_ref[...] + d1_ref[...] + ps
    att = (a0_ref[...] + a1_ref[...] + ps * xb) / den
    out = (jnp.dot(xb, w1_ref[...], preferred_element_type=jnp.float32)
           + jnp.dot(att, w2_ref[...], preferred_element_type=jnp.float32)
           + b2_ref[...])
    nrm = jnp.sqrt(jnp.sum(out * out, axis=1, keepdims=True))
    o_ref[...] = jnp.maximum(out / nrm, 0.0)


def _make_sc_kernel(N, D, S, E, Epad, inv_sqrt_s):
    NC, NS = 2, 16
    NW = NC * NS
    EPW = Epad // NW
    NCHUNK = EPW // _CH
    ROWS_PT = -(-N // (NS * 8)) * 8
    Npad = ROWS_PT * NS
    ZD = Npad // 8
    nz = D // _LANES

    def body(x_hbm, q_hbm, k_hbm, sd_hbm,
             att_out, den_out,
             sd0, sd1, q0, q1, k0, k1, x0, x1, w_v, zb_den,
             att_sh, den_sh, sem0, sem1):
        cid = lax.axis_index("c")
        sid = lax.axis_index("s")
        iota = lax.iota(jnp.int32, _LANES)
        zeros16 = jnp.zeros((_LANES,), jnp.float32)

        def _zero_row(i, c):
            for cc in range(nz):
                x0[i, pl.ds(cc * _LANES, _LANES)] = zeros16
            return c
        lax.fori_loop(0, _CH, _zero_row, 0)

        r0 = sid * ROWS_PT
        off = 0
        rem = ROWS_PT
        while rem > 0:
            n = min(rem, _CH)
            pltpu.sync_copy(x0.at[pl.ds(0, n)],
                            att_sh.at[pl.ds(r0 + off, n)])
            off += n
            rem -= n

        def _zero_den(i, c):
            zb_den[pl.ds(i * _LANES, _LANES)] = zeros16
            return c
        lax.fori_loop(0, ZD // _LANES, _zero_den, 0)

        @pl.when(sid == 0)
        def _():
            for o in range(0, Npad, ZD):
                pltpu.sync_copy(zb_den, den_sh.at[pl.ds(o, ZD)])

        plsc.subcore_barrier()

        wid = cid * NS + sid
        cbase = wid * NCHUNK
        bufs = ((sd0, q0, k0, x0, sem0), (sd1, q1, k1, x1, sem1))

        def start_fetch(b, ch):
            sd_v, q_r, k_r, x_r, sem = bufs[b]
            pltpu.sync_copy(sd_hbm.at[cbase + ch], sd_v)
            pltpu.make_async_copy(q_hbm.at[sd_v.at[0]], q_r, sem).start()
            pltpu.make_async_copy(k_hbm.at[sd_v.at[1]], k_r, sem).start()
            pltpu.make_async_copy(x_hbm.at[sd_v.at[0]], x_r, sem).start()

        def process(b, ch):
            sd_v, q_r, k_r, x_r, sem = bufs[b]
            pltpu.make_async_copy(q_hbm.at[sd_v.at[0]], q_r, sem).wait()
            pltpu.make_async_copy(k_hbm.at[sd_v.at[1]], k_r, sem).wait()
            pltpu.make_async_copy(x_hbm.at[sd_v.at[0]], x_r, sem).wait()
            ebase = (cbase + ch) * _CH
            for g in range(_CH // _LANES):
                row_idx = g * _LANES + iota
                acc = jnp.zeros((_LANES,), jnp.float32)
                for j in range(S):
                    cj = jnp.full((_LANES,), j, jnp.int32)
                    qv = plsc.load_gather(q_r, [row_idx, cj])
                    kv = plsc.load_gather(k_r, [row_idx, cj])
                    acc = acc + qv * kv
                p = jnp.exp(acc * inv_sqrt_s)
                p = jnp.where(ebase + row_idx < E, p, 0.0)
                w_v[pl.ds(g * _LANES, _LANES)] = p
                for l in range(_LANES):
                    ws = p[l]
                    r = g * _LANES + l
                    for cc in range(nz):
                        sl = pl.ds(cc * _LANES, _LANES)
                        x_r[r, sl] = x_r[r, sl] * ws
            pltpu.sync_copy(w_v, den_sh.at[sd_v.at[1]], add=True)
            pltpu.sync_copy(x_r, att_sh.at[sd_v.at[1]], add=True)

        start_fetch(0, 0)
        start_fetch(1, 1)

        def ring(i, c):
            ch0 = i * 2
            process(0, ch0)

            @pl.when(ch0 + 2 < NCHUNK)
            def _():
                start_fetch(0, ch0 + 2)

            process(1, ch0 + 1)

            @pl.when(ch0 + 3 < NCHUNK)
            def _():
                start_fetch(1, ch0 + 3)
            return c

        lax.fori_loop(0, NCHUNK // 2, ring, 0)

        plsc.subcore_barrier()

        pltpu.sync_copy(att_sh.at[pl.ds(r0, ROWS_PT)],
                        att_out.at[cid, pl.ds(r0, ROWS_PT)])

        @pl.when(sid == 0)
        def _():
            pltpu.sync_copy(den_sh, den_out.at[cid])

    mesh = plsc.VectorSubcoreMesh(core_axis_name="c", subcore_axis_name="s")
    return pl.kernel(
        body,
        mesh=mesh,
        compiler_params=pltpu.CompilerParams(
            needs_layout_passes=False, use_tc_tiling_on_sc=False),
        out_type=[
            jax.ShapeDtypeStruct((NC, Npad, D), jnp.float32),
            jax.ShapeDtypeStruct((NC, Npad), jnp.float32),
        ],
        scratch_types=[
            pltpu.VMEM((2, _CH), jnp.int32),
            pltpu.VMEM((2, _CH), jnp.int32),
            pltpu.VMEM((_CH, _LANES), jnp.float32),
            pltpu.VMEM((_CH, _LANES), jnp.float32),
            pltpu.VMEM((_CH, _LANES), jnp.float32),
            pltpu.VMEM((_CH, _LANES), jnp.float32),
            pltpu.VMEM((_CH, D), jnp.float32),
            pltpu.VMEM((_CH, D), jnp.float32),
            pltpu.VMEM((_CH,), jnp.float32),
            pltpu.VMEM((ZD,), jnp.float32),
            pltpu.VMEM_SHARED((Npad, D), jnp.float32),
            pltpu.VMEM_SHARED((Npad,), jnp.float32),
            pltpu.SemaphoreType.DMA,
            pltpu.SemaphoreType.DMA,
        ],
    )


def kernel(x, edge_index, Wq, bq, Wk, bk, W1, W2, b2):
    N, D = x.shape
    S = Wq.shape[0]
    E = edge_index.shape[1]
    inv_sqrt_s = np.float32(1.0 / np.sqrt(S))

    WqT = jnp.zeros((D, _LANES), jnp.float32).at[:, :S].set(Wq.T)
    WkT = jnp.zeros((D, _LANES), jnp.float32).at[:, :S].set(Wk.T)
    bq_p = jnp.zeros((1, _LANES), jnp.float32).at[0, :S].set(bq)
    bk_p = jnp.zeros((1, _LANES), jnp.float32).at[0, :S].set(bk)

    align = 2 * 32 * _CH
    Epad = ((E + align - 1) // align) * align
    pad = Epad - E
    spread = (jnp.arange(pad, dtype=edge_index.dtype) * 8) % N
    src_p = jnp.concatenate([edge_index[0], spread])
    dst_p = jnp.concatenate([edge_index[1], spread])
    sd = jnp.stack([src_p.reshape(Epad // _CH, _CH),
                    dst_p.reshape(Epad // _CH, _CH)], axis=1)

    BA = 400
    grid_a = N // BA
    q_pad, k_pad = pl.pallas_call(
        _proj_body,
        grid=(grid_a,),
        in_specs=[
            pl.BlockSpec((BA, D), lambda i: (i, 0)),
            pl.BlockSpec((D, _LANES), lambda i: (0, 0)),
            pl.BlockSpec((1, _LANES), lambda i: (0, 0)),
            pl.BlockSpec((D, _LANES), lambda i: (0, 0)),
            pl.BlockSpec((1, _LANES), lambda i: (0, 0)),
        ],
        out_specs=[
            pl.BlockSpec((BA, _LANES), lambda i: (i, 0)),
            pl.BlockSpec((BA, _LANES), lambda i: (i, 0)),
        ],
        out_shape=[
            jax.ShapeDtypeStruct((N, _LANES), jnp.float32),
            jax.ShapeDtypeStruct((N, _LANES), jnp.float32),
        ],
    )(x, WqT, bq_p, WkT, bk_p)

    sc_fn = _make_sc_kernel(N, D, S, E, Epad, inv_sqrt_s)
    att_part, den_part = sc_fn(x, q_pad, k_pad, sd)

    a0, a1 = att_part[0, :N], att_part[1, :N]
    d0 = den_part[0, :N].reshape(N, 1)
    d1 = den_part[1, :N].reshape(N, 1)

    out = pl.pallas_call(
        functools.partial(_combine_body, inv_sqrt_s),
        grid=(grid_a,),
        in_specs=[
            pl.BlockSpec((BA, D), lambda i: (i, 0)),
            pl.BlockSpec((BA, _LANES), lambda i: (i, 0)),
            pl.BlockSpec((BA, _LANES), lambda i: (i, 0)),
            pl.BlockSpec((BA, D), lambda i: (i, 0)),
            pl.BlockSpec((BA, D), lambda i: (i, 0)),
            pl.BlockSpec((BA, 1), lambda i: (i, 0)),
            pl.BlockSpec((BA, 1), lambda i: (i, 0)),
            pl.BlockSpec((D, D), lambda i: (0, 0)),
            pl.BlockSpec((D, D), lambda i: (0, 0)),
            pl.BlockSpec((1, D), lambda i: (0, 0)),
        ],
        out_specs=pl.BlockSpec((BA, D), lambda i: (i, 0)),
        out_shape=jax.ShapeDtypeStruct((N, D), jnp.float32),
    )(x, q_pad, k_pad, a0, a1, d0, d1, W1.T, W2.T, b2.reshape(1, D))
    return out


# speedup vs baseline: 22.6183x; 1.0372x over previous
import functools

import numpy as np
import jax
import jax.numpy as jnp
from jax import lax
from jax.experimental import pallas as pl
from jax.experimental.pallas import tpu as pltpu
from jax.experimental.pallas import tpu_sc as plsc


_LANES = 16
_CH = 128


def _proj_body(x_ref, wq_ref, bq_ref, wk_ref, bk_ref, q_ref, k_ref):
    xb = x_ref[...]
    q_ref[...] = jnp.tanh(
        jnp.dot(xb, wq_ref[...], preferred_element_type=jnp.float32) + bq_ref[...])
    k_ref[...] = (
        jnp.dot(xb, wk_ref[...], preferred_element_type=jnp.float32) + bk_ref[...])


def _combine_body(inv_sqrt_s, x_ref, q_ref, k_ref, a_ref, d_ref,
                  w1_ref, w2_ref, b2_ref, o_ref):
    xb = x_ref[...]
    ps = jnp.exp(jnp.sum(q_ref[...] * k_ref[...], axis=1, keepdims=True)
                 * inv_sqrt_s)
    den = d_ref[0, 0] + d_ref[0, 1] + ps[:, 0]
    att = (a_ref[0] + a_ref[1] + ps * xb) / den[:, None]
    out = (jnp.dot(xb, w1_ref[...], preferred_element_type=jnp.float32)
           + jnp.dot(att, w2_ref[...], preferred_element_type=jnp.float32)
           + b2_ref[...])
    nrm = jnp.sqrt(jnp.sum(out * out, axis=1, keepdims=True))
    o_ref[...] = jnp.maximum(out / nrm, 0.0)


def _make_sc_kernel(N, D, S, E, Epad, inv_sqrt_s):
    NC, NS = 2, 16
    NW = NC * NS
    EPW = Epad // NW
    NCHUNK = EPW // _CH
    ROWS_PT = -(-N // (NS * 8)) * 8
    Npad = ROWS_PT * NS
    ZD = Npad // 8
    nz = D // _LANES

    def body(x_hbm, q_hbm, k_hbm, sd_hbm,
             att_out, den_out,
             sd0, sd1, q0, q1, k0, k1, x0, x1, w0, w1, zb_den,
             att_sh, den_sh, sem0, sem1, ssem0, ssem1):
        cid = lax.axis_index("c")
        sid = lax.axis_index("s")
        iota = lax.iota(jnp.int32, _LANES)
        zeros16 = jnp.zeros((_LANES,), jnp.float32)

        def _zero_row(i, c):
            for cc in range(nz):
                x0[i, pl.ds(cc * _LANES, _LANES)] = zeros16
            return c
        lax.fori_loop(0, _CH, _zero_row, 0)

        r0 = sid * ROWS_PT
        off = 0
        rem = ROWS_PT
        while rem > 0:
            n = min(rem, _CH)
            pltpu.sync_copy(x0.at[pl.ds(0, n)],
                            att_sh.at[pl.ds(r0 + off, n)])
            off += n
            rem -= n

        def _zero_den(i, c):
            zb_den[pl.ds(i * _LANES, _LANES)] = zeros16
            return c
        lax.fori_loop(0, ZD // _LANES, _zero_den, 0)

        @pl.when(sid == 0)
        def _():
            for o in range(0, Npad, ZD):
                pltpu.sync_copy(zb_den, den_sh.at[pl.ds(o, ZD)])

        plsc.subcore_barrier()

        wid = cid * NS + sid
        cbase = wid * NCHUNK
        bufs = ((sd0, q0, k0, x0, w0, sem0, ssem0),
                (sd1, q1, k1, x1, w1, sem1, ssem1))

        def start_fetch(b, ch):
            sd_v, q_r, k_r, x_r, w_v, sem, _ = bufs[b]
            pltpu.sync_copy(sd_hbm.at[cbase + ch], sd_v)
            pltpu.make_async_copy(q_hbm.at[sd_v.at[0]], q_r, sem).start()
            pltpu.make_async_copy(k_hbm.at[sd_v.at[1]], k_r, sem).start()
            pltpu.make_async_copy(x_hbm.at[sd_v.at[0]], x_r, sem).start()

        def compute(b, ch):
            sd_v, q_r, k_r, x_r, w_v, sem, ssem = bufs[b]
            pltpu.make_async_copy(q_hbm.at[sd_v.at[0]], q_r, sem).wait()
            pltpu.make_async_copy(k_hbm.at[sd_v.at[1]], k_r, sem).wait()
            pltpu.make_async_copy(x_hbm.at[sd_v.at[0]], x_r, sem).wait()
            ebase = (cbase + ch) * _CH
            for g in range(_CH // _LANES):
                row_idx = g * _LANES + iota
                acc = jnp.zeros((_LANES,), jnp.float32)
                for j in range(S):
                    cj = jnp.full((_LANES,), j, jnp.int32)
                    qv = plsc.load_gather(q_r, [row_idx, cj])
                    kv = plsc.load_gather(k_r, [row_idx, cj])
                    acc = acc + qv * kv
                p = jnp.exp(acc * inv_sqrt_s)
                p = jnp.where(ebase + row_idx < E, p, 0.0)
                w_v[pl.ds(g * _LANES, _LANES)] = p
                for l in range(_LANES):
                    ws = p[l]
                    r = g * _LANES + l
                    for cc in range(nz):
                        sl = pl.ds(cc * _LANES, _LANES)
                        x_r[r, sl] = x_r[r, sl] * ws
            pltpu.make_async_copy(w_v, den_sh.at[sd_v.at[1]], ssem).start(
                add=True)
            pltpu.make_async_copy(x_r, att_sh.at[sd_v.at[1]], ssem).start(
                add=True)

        def wait_scatter(b):
            sd_v, q_r, k_r, x_r, w_v, sem, ssem = bufs[b]
            pltpu.make_async_copy(w_v, den_sh.at[sd_v.at[1]], ssem).wait()
            pltpu.make_async_copy(x_r, att_sh.at[sd_v.at[1]], ssem).wait()

        start_fetch(0, 0)
        start_fetch(1, 1)

        def ring(i, c):
            ch0 = i * 2
            compute(0, ch0)
            compute(1, ch0 + 1)
            wait_scatter(0)

            @pl.when(ch0 + 2 < NCHUNK)
            def _():
                start_fetch(0, ch0 + 2)

            wait_scatter(1)

            @pl.when(ch0 + 3 < NCHUNK)
            def _():
                start_fetch(1, ch0 + 3)
            return c

        lax.fori_loop(0, NCHUNK // 2, ring, 0)

        plsc.subcore_barrier()

        pltpu.sync_copy(att_sh.at[pl.ds(r0, ROWS_PT)],
                        att_out.at[cid, pl.ds(r0, ROWS_PT)])

        @pl.when(sid == 0)
        def _():
            pltpu.sync_copy(den_sh, den_out.at[cid])

    mesh = plsc.VectorSubcoreMesh(core_axis_name="c", subcore_axis_name="s")
    return pl.kernel(
        body,
        mesh=mesh,
        compiler_params=pltpu.CompilerParams(
            needs_layout_passes=False, use_tc_tiling_on_sc=False),
        out_type=[
            jax.ShapeDtypeStruct((NC, Npad, D), jnp.float32),
            jax.ShapeDtypeStruct((NC, Npad), jnp.float32),
        ],
        scratch_types=[
            pltpu.VMEM((2, _CH), jnp.int32),
            pltpu.VMEM((2, _CH), jnp.int32),
            pltpu.VMEM((_CH, _LANES), jnp.float32),
            pltpu.VMEM((_CH, _LANES), jnp.float32),
            pltpu.VMEM((_CH, _LANES), jnp.float32),
            pltpu.VMEM((_CH, _LANES), jnp.float32),
            pltpu.VMEM((_CH, D), jnp.float32),
            pltpu.VMEM((_CH, D), jnp.float32),
            pltpu.VMEM((_CH,), jnp.float32),
            pltpu.VMEM((_CH,), jnp.float32),
            pltpu.VMEM((ZD,), jnp.float32),
            pltpu.VMEM_SHARED((Npad, D), jnp.float32),
            pltpu.VMEM_SHARED((Npad,), jnp.float32),
            pltpu.SemaphoreType.DMA,
            pltpu.SemaphoreType.DMA,
            pltpu.SemaphoreType.DMA,
            pltpu.SemaphoreType.DMA,
        ],
    )


def kernel(x, edge_index, Wq, bq, Wk, bk, W1, W2, b2):
    N, D = x.shape
    S = Wq.shape[0]
    E = edge_index.shape[1]
    inv_sqrt_s = np.float32(1.0 / np.sqrt(S))

    WqT = jnp.zeros((D, _LANES), jnp.float32).at[:, :S].set(Wq.T)
    WkT = jnp.zeros((D, _LANES), jnp.float32).at[:, :S].set(Wk.T)
    bq_p = jnp.zeros((1, _LANES), jnp.float32).at[0, :S].set(bq)
    bk_p = jnp.zeros((1, _LANES), jnp.float32).at[0, :S].set(bk)

    align = 2 * 32 * _CH
    Epad = ((E + align - 1) // align) * align
    pad = Epad - E
    spread = (jnp.arange(pad, dtype=edge_index.dtype) * 8) % N
    src_p = jnp.concatenate([edge_index[0], spread])
    dst_p = jnp.concatenate([edge_index[1], spread])
    sd = jnp.stack([src_p.reshape(Epad // _CH, _CH),
                    dst_p.reshape(Epad // _CH, _CH)], axis=1)

    BA = 400
    grid_a = N // BA
    q_pad, k_pad = pl.pallas_call(
        _proj_body,
        grid=(grid_a,),
        in_specs=[
            pl.BlockSpec((BA, D), lambda i: (i, 0)),
            pl.BlockSpec((D, _LANES), lambda i: (0, 0)),
            pl.BlockSpec((1, _LANES), lambda i: (0, 0)),
            pl.BlockSpec((D, _LANES), lambda i: (0, 0)),
            pl.BlockSpec((1, _LANES), lambda i: (0, 0)),
        ],
        out_specs=[
            pl.BlockSpec((BA, _LANES), lambda i: (i, 0)),
            pl.BlockSpec((BA, _LANES), lambda i: (i, 0)),
        ],
        out_shape=[
            jax.ShapeDtypeStruct((N, _LANES), jnp.float32),
            jax.ShapeDtypeStruct((N, _LANES), jnp.float32),
        ],
    )(x, WqT, bq_p, WkT, bk_p)

    sc_fn = _make_sc_kernel(N, D, S, E, Epad, inv_sqrt_s)
    att_part, den_part = sc_fn(x, q_pad, k_pad, sd)

    den3 = den_part[:, :N].reshape(2, grid_a, BA).transpose(1, 0, 2)
    out = pl.pallas_call(
        functools.partial(_combine_body, inv_sqrt_s),
        grid=(grid_a,),
        in_specs=[
            pl.BlockSpec((BA, D), lambda i: (i, 0)),
            pl.BlockSpec((BA, _LANES), lambda i: (i, 0)),
            pl.BlockSpec((BA, _LANES), lambda i: (i, 0)),
            pl.BlockSpec((2, BA, D), lambda i: (0, i, 0)),
            pl.BlockSpec((1, 2, BA), lambda i: (i, 0, 0)),
            pl.BlockSpec((D, D), lambda i: (0, 0)),
            pl.BlockSpec((D, D), lambda i: (0, 0)),
            pl.BlockSpec((1, D), lambda i: (0, 0)),
        ],
        out_specs=pl.BlockSpec((BA, D), lambda i: (i, 0)),
        out_shape=jax.ShapeDtypeStruct((N, D), jnp.float32),
    )(x, q_pad, k_pad, att_part, den3, W1.T, W2.T, b2.reshape(1, D))
    return out
